# Initial kernel scaffold; baseline (speedup 1.0000x reference)
import jax
import jax.numpy as jnp
from jax.experimental import pallas as pl


def kernel(R, S, H, node_data, Ra_data, W, Omega, W_e, Omega_e, B_ne, B_en, P3, V0_w, V0_b, V1_w, V1_b):
    raise NotImplementedError("write your pallas kernel here")


# speedup vs baseline: 2.5748x; 2.5748x over previous
import functools

import jax
import jax.numpy as jnp
from jax import lax
from jax.experimental import pallas as pl
from jax.experimental.pallas import tpu as pltpu
from jax.experimental.pallas import tpu_sc as plsc

_N = 10000
_E = 320000
_NH = 128
_NE = 16
_KAPPA = 0.9
_ITERS = 5

_NC = 2
_NS = 16
_NW = _NC * _NS
_CH = 512
_NCHUNK = _E // _CH
_IPC = _CH // 128
_TROWS = _N // _NS

_HI = lax.Precision.HIGHEST


_NHH = _NH // _NC


def _sc_edge_body(r2_hbm, s2_hbm, h2_hbm, z_hbm, y_hbm, a_hbm,
                  he_hbm, acc128_hbm, acc16_hbm,
                  idx_r, idx_s, hbuf, zr, zs, abuf, hebuf, ybuf,
                  gsem, acc128_sh, acc16_sh):
    c = lax.axis_index("c")
    s = lax.axis_index("s")
    w = s * _NC + c
    base_n = s * _TROWS

    def _zero_body(e, _):
        hebuf[e, :] = jnp.zeros((16,), jnp.float32)
        for v in range(_NHH // 16):
            ybuf[e, pl.ds(v * 16, 16)] = jnp.zeros((16,), jnp.float32)
        return 0
    lax.fori_loop(0, _CH, _zero_body, 0)
    pltpu.sync_copy(ybuf, acc128_sh.at[pl.ds(base_n, _CH)])
    pltpu.sync_copy(ybuf.at[pl.ds(0, _TROWS - _CH)],
                    acc128_sh.at[pl.ds(base_n + _CH, _TROWS - _CH)])
    pltpu.sync_copy(hebuf, acc16_sh.at[pl.ds(base_n, _CH)])
    pltpu.sync_copy(hebuf.at[pl.ds(0, _TROWS - _CH)],
                    acc16_sh.at[pl.ds(base_n + _CH, _TROWS - _CH)])
    plsc.subcore_barrier()

    nchunks1 = (_NCHUNK - w + _NW - 1) // _NW

    def _round1(r, _):
        chunk = w + r * _NW
        cbase = chunk * _CH
        pltpu.sync_copy(r2_hbm.at[chunk], idx_r)
        pltpu.sync_copy(s2_hbm.at[chunk], idx_s)
        pltpu.sync_copy(a_hbm.at[pl.ds(cbase, _CH)], abuf)
        for j in range(_IPC):
            pltpu.async_copy(z_hbm.at[idx_r.at[j]],
                             zr.at[pl.ds(j * 128, 128)], gsem).wait()
            pltpu.async_copy(z_hbm.at[idx_s.at[j]],
                             zs.at[pl.ds(j * 128, 128)], gsem).wait()

        def _he(e, _):
            row = abuf[e, :] + zr[e, :] + zs[e, :]
            hebuf[e, :] = jnp.maximum(row, 0.0)
            return 0
        lax.fori_loop(0, _CH, _he, 0)
        pltpu.sync_copy(hebuf, he_hbm.at[pl.ds(cbase, _CH)])
        for j in range(_IPC):
            pltpu.sync_copy(hebuf.at[pl.ds(j * 128, 128)],
                            acc16_sh.at[idx_s.at[j]], add=True)
        return 0

    lax.fori_loop(0, nchunks1, _round1, 0)

    nchunks2 = (_NCHUNK - s + _NS - 1) // _NS
    yh_hbm = y_hbm.at[c]

    def _round2(r, _):
        chunk = s + r * _NS
        pltpu.sync_copy(r2_hbm.at[chunk], idx_r)
        pltpu.sync_copy(s2_hbm.at[chunk], idx_s)
        pltpu.sync_copy(h2_hbm.at[chunk], hbuf)
        for j in range(_IPC):
            pltpu.async_copy(yh_hbm.at[idx_r.at[j]],
                             ybuf.at[pl.ds(j * 128, 128)], gsem).wait()

        for j in range(_IPC):
            def _scale(g, _, j=j):
                h16 = hbuf[j, pl.ds(g * 16, 16)]
                for k in range(16):
                    e = j * 128 + g * 16 + k
                    hv = jnp.full((16,), h16[k], jnp.float32)
                    for v in range(_NHH // 16):
                        sl = pl.ds(v * 16, 16)
                        ybuf[e, sl] = ybuf[e, sl] * hv
                return 0
            lax.fori_loop(0, 8, _scale, 0)

        for j in range(_IPC):
            pltpu.sync_copy(ybuf.at[pl.ds(j * 128, 128)],
                            acc128_sh.at[idx_s.at[j]], add=True)
        return 0

    lax.fori_loop(0, nchunks2, _round2, 0)

    plsc.subcore_barrier()
    pltpu.sync_copy(acc128_sh.at[pl.ds(base_n, _TROWS)],
                    acc128_hbm.at[c, s])
    pltpu.sync_copy(acc16_sh.at[pl.ds(base_n, _TROWS)],
                    acc16_hbm.at[c, s])


_sc_edge = pl.kernel(
    _sc_edge_body,
    out_type=[
        jax.ShapeDtypeStruct((_E, _NE), jnp.float32),
        jax.ShapeDtypeStruct((_NC, _NS, _TROWS, _NHH), jnp.float32),
        jax.ShapeDtypeStruct((_NC, _NS, _TROWS, _NE), jnp.float32),
    ],
    mesh=plsc.VectorSubcoreMesh(core_axis_name="c", subcore_axis_name="s"),
    compiler_params=pltpu.CompilerParams(use_tc_tiling_on_sc=False),
    scratch_types=[
        pltpu.VMEM((_IPC, 128), jnp.int32),
        pltpu.VMEM((_IPC, 128), jnp.int32),
        pltpu.VMEM((_IPC, 128), jnp.float32),
        pltpu.VMEM((_CH, _NE), jnp.float32),
        pltpu.VMEM((_CH, _NE), jnp.float32),
        pltpu.VMEM((_CH, _NE), jnp.float32),
        pltpu.VMEM((_CH, _NE), jnp.float32),
        pltpu.VMEM((_CH, _NHH), jnp.float32),
        pltpu.SemaphoreType.DMA,
        pltpu.VMEM_SHARED((_N, _NHH), jnp.float32),
        pltpu.VMEM_SHARED((_N, _NE), jnp.float32),
    ],
)


def _u_body(nd_ref, om_ref, u_ref):
    u_ref[...] = lax.dot_general(
        nd_ref[...], om_ref[...], (((0,), (1,)), ((), ())),
        precision=_HI, preferred_element_type=jnp.float32)


def _tc_u(node_data, Omega):
    return pl.pallas_call(
        _u_body,
        out_shape=jax.ShapeDtypeStruct((_N, _NH), jnp.float32),
    )(node_data, Omega)


_BE = 6400


def _ue_body(ra_ref, oe_ref, ue_ref):
    ue_ref[...] = lax.dot_general(
        ra_ref[...], oe_ref[...], (((0,), (1,)), ((), ())),
        precision=_HI, preferred_element_type=jnp.float32)


def _tc_ue(Ra_data, Omega_e):
    return pl.pallas_call(
        _ue_body,
        grid=(_E // _BE,),
        in_specs=[
            pl.BlockSpec((_NE, _BE), lambda i: (0, i)),
            pl.BlockSpec((_NE, _NE), lambda i: (0, 0)),
        ],
        out_specs=pl.BlockSpec((_BE, _NE), lambda i: (i, 0)),
        out_shape=jax.ShapeDtypeStruct((_E, _NE), jnp.float32),
    )(Ra_data, Omega_e)


def _ea_body(he_ref, ue_ref, we_ref, a_ref):
    a_ref[...] = _KAPPA * lax.dot_general(
        he_ref[...], we_ref[...], (((1,), (1,)), ((), ())),
        precision=_HI, preferred_element_type=jnp.float32) + ue_ref[...]


def _tc_edgea(He, Ue, W_e):
    return pl.pallas_call(
        _ea_body,
        grid=(_E // _BE,),
        in_specs=[
            pl.BlockSpec((_BE, _NE), lambda i: (i, 0)),
            pl.BlockSpec((_BE, _NE), lambda i: (i, 0)),
            pl.BlockSpec((_NE, _NE), lambda i: (0, 0)),
        ],
        out_specs=pl.BlockSpec((_BE, _NE), lambda i: (i, 0)),
        out_shape=jax.ShapeDtypeStruct((_E, _NE), jnp.float32),
    )(He, Ue, W_e)


_BN = 2000


def _upd_body(a128_ref, a16_ref, u_ref, w_ref, bne_ref, ben_ref,
              x_ref, y_ref, z_ref):
    acc = jnp.concatenate([a128_ref[0], a128_ref[1]], axis=1)
    e2n = a16_ref[0] + a16_ref[1]
    x = jnp.maximum(
        acc + lax.dot_general(e2n, ben_ref[...], (((1,), (1,)), ((), ())),
                              precision=_HI,
                              preferred_element_type=jnp.float32)
        + u_ref[...], 0.0)
    x_ref[...] = x
    for cc in range(_NC):
        wh = w_ref[pl.ds(cc * _NHH, _NHH), :]
        y_ref[cc] = _KAPPA * lax.dot_general(
            x, wh, (((1,), (1,)), ((), ())),
            precision=_HI, preferred_element_type=jnp.float32)
    z_ref[...] = lax.dot_general(
        x, bne_ref[...], (((1,), (1,)), ((), ())),
        precision=_HI, preferred_element_type=jnp.float32)


def _tc_update(a128, a16, U, W, B_ne, B_en):
    return pl.pallas_call(
        _upd_body,
        grid=(_N // _BN,),
        in_specs=[
            pl.BlockSpec((_NC, _BN, _NHH), lambda i: (0, i, 0)),
            pl.BlockSpec((_NC, _BN, _NE), lambda i: (0, i, 0)),
            pl.BlockSpec((_BN, _NH), lambda i: (i, 0)),
            pl.BlockSpec((_NH, _NH), lambda i: (0, 0)),
            pl.BlockSpec((_NE, _NH), lambda i: (0, 0)),
            pl.BlockSpec((_NH, _NE), lambda i: (0, 0)),
        ],
        out_specs=[
            pl.BlockSpec((_BN, _NH), lambda i: (i, 0)),
            pl.BlockSpec((_NC, _BN, _NHH), lambda i: (0, i, 0)),
            pl.BlockSpec((_BN, _NE), lambda i: (i, 0)),
        ],
        out_shape=[
            jax.ShapeDtypeStruct((_N, _NH), jnp.float32),
            jax.ShapeDtypeStruct((_NC, _N, _NHH), jnp.float32),
            jax.ShapeDtypeStruct((_N, _NE), jnp.float32),
        ],
    )(a128, a16, U, W, B_ne, B_en)


def _ro_body(x_ref, v0w_ref, v0b_ref, v1w_ref, v1b_ref, o_ref):
    hdd = jnp.maximum(
        lax.dot_general(x_ref[...], v0w_ref[...], (((1,), (1,)), ((), ())),
                        precision=_HI, preferred_element_type=jnp.float32)
        + v0b_ref[...][None, :], 0.0)
    o_ref[...] = lax.dot_general(
        hdd, v1w_ref[...], (((1,), (1,)), ((), ())),
        precision=_HI, preferred_element_type=jnp.float32) \
        + v1b_ref[...][None, :]


def _tc_readout(X, V0_w, V0_b, V1_w, V1_b):
    return pl.pallas_call(
        _ro_body,
        grid=(_N // _BN,),
        in_specs=[
            pl.BlockSpec((_BN, _NH), lambda i: (i, 0)),
            pl.BlockSpec((_NH, _NH), lambda i: (0, 0)),
            pl.BlockSpec((_NH,), lambda i: (0,)),
            pl.BlockSpec((_NH, _NH), lambda i: (0, 0)),
            pl.BlockSpec((_NH,), lambda i: (0,)),
        ],
        out_specs=pl.BlockSpec((_BN, _NH), lambda i: (i, 0)),
        out_shape=jax.ShapeDtypeStruct((_N, _NH), jnp.float32),
    )(X, V0_w, V0_b, V1_w, V1_b)


def _lg_body(he_ref, p3_ref, o_ref):
    o_ref[...] = lax.dot_general(
        he_ref[...], p3_ref[...], (((1,), (1,)), ((), ())),
        precision=_HI, preferred_element_type=jnp.float32)


def _tc_logits(He, P3):
    return pl.pallas_call(
        _lg_body,
        grid=(_E // _BE,),
        in_specs=[
            pl.BlockSpec((_BE, _NE), lambda i: (i, 0)),
            pl.BlockSpec((3, _NE), lambda i: (0, 0)),
        ],
        out_specs=pl.BlockSpec((_BE, 3), lambda i: (i, 0)),
        out_shape=jax.ShapeDtypeStruct((_E, 3), jnp.float32),
    )(He, P3)


def _ht_body(he_ref, o_ref):
    o_ref[...] = he_ref[...].T


def _tc_het(He):
    return pl.pallas_call(
        _ht_body,
        grid=(_E // _BE,),
        in_specs=[pl.BlockSpec((_BE, _NE), lambda i: (i, 0))],
        out_specs=pl.BlockSpec((_NE, _BE), lambda i: (0, i)),
        out_shape=jax.ShapeDtypeStruct((_NE, _E), jnp.float32),
    )(He)


def kernel(R, S, H, node_data, Ra_data, W, Omega, W_e, Omega_e,
           B_ne, B_en, P3, V0_w, V0_b, V1_w, V1_b):
    r2 = R.reshape(_NCHUNK, _IPC, 128)
    s2 = S.reshape(_NCHUNK, _IPC, 128)
    h2 = H.reshape(_NCHUNK, _IPC, 128)

    U = _tc_u(node_data, Omega)
    Ue = _tc_ue(Ra_data, Omega_e)

    He = jnp.zeros((_E, _NE), jnp.float32)
    Y = jnp.zeros((_NC, _N, _NHH), jnp.float32)
    Z = jnp.zeros((_N, _NE), jnp.float32)
    X = jnp.zeros((_N, _NH), jnp.float32)
    for _ in range(_ITERS):
        A = _tc_edgea(He, Ue, W_e)
        He, a128, a16 = _sc_edge(r2, s2, h2, Z, Y, A)
        X, Y, Z = _tc_update(a128.reshape(_NC, _N, _NHH),
                             a16.reshape(_NC, _N, _NE), U, W, B_ne, B_en)

    x = _tc_readout(X, V0_w, V0_b, V1_w, V1_b)
    logits = _tc_logits(He, P3)
    He_T = _tc_het(He)
    return (x, He_T, logits)


# speedup vs baseline: 3.0051x; 1.1671x over previous
import functools

import jax
import jax.numpy as jnp
from jax import lax
from jax.experimental import pallas as pl
from jax.experimental.pallas import tpu as pltpu
from jax.experimental.pallas import tpu_sc as plsc

_N = 10000
_E = 320000
_NH = 128
_NE = 16
_KAPPA = 0.9
_ITERS = 5

_NC = 2
_NS = 16
_NW = _NC * _NS
_CH = 512
_NCHUNK = _E // _CH
_IPC = _CH // 128
_TROWS = _N // _NS

_HI = lax.Precision.HIGHEST


_NHH = _NH // _NC


def _sc_edge_body(r2_hbm, s2_hbm, h2_hbm, z_hbm, y_hbm, a_hbm,
                  he_hbm, acc128_hbm, acc16_hbm,
                  idx_r, idx_s, hbuf, zr, zs, abuf, hebuf, ybuf,
                  gsem, acc128_sh, acc16_sh):
    c = lax.axis_index("c")
    s = lax.axis_index("s")
    w = s * _NC + c
    base_n = s * _TROWS

    def _zero_body(e, _):
        hebuf[e, :] = jnp.zeros((16,), jnp.float32)
        for v in range(_NHH // 16):
            ybuf[e, pl.ds(v * 16, 16)] = jnp.zeros((16,), jnp.float32)
        return 0
    lax.fori_loop(0, _CH, _zero_body, 0)
    pltpu.sync_copy(ybuf, acc128_sh.at[pl.ds(base_n, _CH)])
    pltpu.sync_copy(ybuf.at[pl.ds(0, _TROWS - _CH)],
                    acc128_sh.at[pl.ds(base_n + _CH, _TROWS - _CH)])
    pltpu.sync_copy(hebuf, acc16_sh.at[pl.ds(base_n, _CH)])
    pltpu.sync_copy(hebuf.at[pl.ds(0, _TROWS - _CH)],
                    acc16_sh.at[pl.ds(base_n + _CH, _TROWS - _CH)])
    plsc.subcore_barrier()

    nchunks1 = (_NCHUNK - w + _NW - 1) // _NW

    def _round1(r, _):
        chunk = w + r * _NW
        cbase = chunk * _CH
        pltpu.sync_copy(r2_hbm.at[chunk], idx_r)
        pltpu.sync_copy(s2_hbm.at[chunk], idx_s)
        pltpu.sync_copy(a_hbm.at[pl.ds(cbase, _CH)], abuf)
        cps = []
        for j in range(_IPC):
            cps.append(pltpu.async_copy(z_hbm.at[idx_r.at[j]],
                                        zr.at[pl.ds(j * 128, 128)], gsem))
            cps.append(pltpu.async_copy(z_hbm.at[idx_s.at[j]],
                                        zs.at[pl.ds(j * 128, 128)], gsem))
        for cp in cps:
            cp.wait()

        def _he(e4, _):
            for k in range(4):
                e = e4 * 4 + k
                row = abuf[e, :] + zr[e, :] + zs[e, :]
                hebuf[e, :] = jnp.maximum(row, 0.0)
            return 0
        lax.fori_loop(0, _CH // 4, _he, 0)
        pltpu.sync_copy(hebuf, he_hbm.at[pl.ds(cbase, _CH)])
        for j in range(_IPC):
            pltpu.sync_copy(hebuf.at[pl.ds(j * 128, 128)],
                            acc16_sh.at[idx_s.at[j]], add=True)
        return 0

    lax.fori_loop(0, nchunks1, _round1, 0)

    nchunks2 = (_NCHUNK - s + _NS - 1) // _NS
    yh_hbm = y_hbm.at[c]

    def _round2(r, _):
        chunk = s + r * _NS
        pltpu.sync_copy(r2_hbm.at[chunk], idx_r)
        pltpu.sync_copy(s2_hbm.at[chunk], idx_s)
        pltpu.sync_copy(h2_hbm.at[chunk], hbuf)
        cps = []
        for j in range(_IPC):
            cps.append(pltpu.async_copy(yh_hbm.at[idx_r.at[j]],
                                        ybuf.at[pl.ds(j * 128, 128)], gsem))
        for cp in cps:
            cp.wait()

        for j in range(_IPC):
            def _scale(g, _, j=j):
                h16 = hbuf[j, pl.ds(g * 16, 16)]
                for k in range(16):
                    e = j * 128 + g * 16 + k
                    hv = jnp.full((16,), h16[k], jnp.float32)
                    for v in range(_NHH // 16):
                        sl = pl.ds(v * 16, 16)
                        ybuf[e, sl] = ybuf[e, sl] * hv
                return 0
            lax.fori_loop(0, 8, _scale, 0)

        for j in range(_IPC):
            pltpu.sync_copy(ybuf.at[pl.ds(j * 128, 128)],
                            acc128_sh.at[idx_s.at[j]], add=True)
        return 0

    lax.fori_loop(0, nchunks2, _round2, 0)

    plsc.subcore_barrier()
    pltpu.sync_copy(acc128_sh.at[pl.ds(base_n, _TROWS)],
                    acc128_hbm.at[c, s])
    pltpu.sync_copy(acc16_sh.at[pl.ds(base_n, _TROWS)],
                    acc16_hbm.at[c, s])


_sc_edge = pl.kernel(
    _sc_edge_body,
    out_type=[
        jax.ShapeDtypeStruct((_E, _NE), jnp.float32),
        jax.ShapeDtypeStruct((_NC, _NS, _TROWS, _NHH), jnp.float32),
        jax.ShapeDtypeStruct((_NC, _NS, _TROWS, _NE), jnp.float32),
    ],
    mesh=plsc.VectorSubcoreMesh(core_axis_name="c", subcore_axis_name="s"),
    compiler_params=pltpu.CompilerParams(use_tc_tiling_on_sc=False),
    scratch_types=[
        pltpu.VMEM((_IPC, 128), jnp.int32),
        pltpu.VMEM((_IPC, 128), jnp.int32),
        pltpu.VMEM((_IPC, 128), jnp.float32),
        pltpu.VMEM((_CH, _NE), jnp.float32),
        pltpu.VMEM((_CH, _NE), jnp.float32),
        pltpu.VMEM((_CH, _NE), jnp.float32),
        pltpu.VMEM((_CH, _NE), jnp.float32),
        pltpu.VMEM((_CH, _NHH), jnp.float32),
        pltpu.SemaphoreType.DMA,
        pltpu.VMEM_SHARED((_N, _NHH), jnp.float32),
        pltpu.VMEM_SHARED((_N, _NE), jnp.float32),
    ],
)


def _u_body(nd_ref, om_ref, u_ref):
    u_ref[...] = lax.dot_general(
        nd_ref[...], om_ref[...], (((0,), (1,)), ((), ())),
        precision=_HI, preferred_element_type=jnp.float32)


def _tc_u(node_data, Omega):
    return pl.pallas_call(
        _u_body,
        out_shape=jax.ShapeDtypeStruct((_N, _NH), jnp.float32),
    )(node_data, Omega)


_BE = 6400


def _ue_body(ra_ref, oe_ref, ue_ref):
    ue_ref[...] = lax.dot_general(
        ra_ref[...], oe_ref[...], (((0,), (1,)), ((), ())),
        precision=_HI, preferred_element_type=jnp.float32)


def _tc_ue(Ra_data, Omega_e):
    return pl.pallas_call(
        _ue_body,
        grid=(_E // _BE,),
        in_specs=[
            pl.BlockSpec((_NE, _BE), lambda i: (0, i)),
            pl.BlockSpec((_NE, _NE), lambda i: (0, 0)),
        ],
        out_specs=pl.BlockSpec((_BE, _NE), lambda i: (i, 0)),
        out_shape=jax.ShapeDtypeStruct((_E, _NE), jnp.float32),
    )(Ra_data, Omega_e)


def _ea_body(he_ref, ue_ref, we_ref, a_ref):
    a_ref[...] = _KAPPA * lax.dot_general(
        he_ref[...], we_ref[...], (((1,), (1,)), ((), ())),
        precision=_HI, preferred_element_type=jnp.float32) + ue_ref[...]


def _tc_edgea(He, Ue, W_e):
    return pl.pallas_call(
        _ea_body,
        grid=(_E // _BE,),
        in_specs=[
            pl.BlockSpec((_BE, _NE), lambda i: (i, 0)),
            pl.BlockSpec((_BE, _NE), lambda i: (i, 0)),
            pl.BlockSpec((_NE, _NE), lambda i: (0, 0)),
        ],
        out_specs=pl.BlockSpec((_BE, _NE), lambda i: (i, 0)),
        out_shape=jax.ShapeDtypeStruct((_E, _NE), jnp.float32),
    )(He, Ue, W_e)


_BN = 2000


def _upd_body(a128_ref, a16_ref, u_ref, w_ref, bne_ref, ben_ref,
              x_ref, y_ref, z_ref):
    acc = jnp.concatenate([a128_ref[0], a128_ref[1]], axis=1)
    e2n = a16_ref[0] + a16_ref[1]
    x = jnp.maximum(
        acc + lax.dot_general(e2n, ben_ref[...], (((1,), (1,)), ((), ())),
                              precision=_HI,
                              preferred_element_type=jnp.float32)
        + u_ref[...], 0.0)
    x_ref[...] = x
    for cc in range(_NC):
        wh = w_ref[pl.ds(cc * _NHH, _NHH), :]
        y_ref[cc] = _KAPPA * lax.dot_general(
            x, wh, (((1,), (1,)), ((), ())),
            precision=_HI, preferred_element_type=jnp.float32)
    z_ref[...] = lax.dot_general(
        x, bne_ref[...], (((1,), (1,)), ((), ())),
        precision=_HI, preferred_element_type=jnp.float32)


def _tc_update(a128, a16, U, W, B_ne, B_en):
    return pl.pallas_call(
        _upd_body,
        grid=(_N // _BN,),
        in_specs=[
            pl.BlockSpec((_NC, _BN, _NHH), lambda i: (0, i, 0)),
            pl.BlockSpec((_NC, _BN, _NE), lambda i: (0, i, 0)),
            pl.BlockSpec((_BN, _NH), lambda i: (i, 0)),
            pl.BlockSpec((_NH, _NH), lambda i: (0, 0)),
            pl.BlockSpec((_NE, _NH), lambda i: (0, 0)),
            pl.BlockSpec((_NH, _NE), lambda i: (0, 0)),
        ],
        out_specs=[
            pl.BlockSpec((_BN, _NH), lambda i: (i, 0)),
            pl.BlockSpec((_NC, _BN, _NHH), lambda i: (0, i, 0)),
            pl.BlockSpec((_BN, _NE), lambda i: (i, 0)),
        ],
        out_shape=[
            jax.ShapeDtypeStruct((_N, _NH), jnp.float32),
            jax.ShapeDtypeStruct((_NC, _N, _NHH), jnp.float32),
            jax.ShapeDtypeStruct((_N, _NE), jnp.float32),
        ],
    )(a128, a16, U, W, B_ne, B_en)


def _ro_body(x_ref, v0w_ref, v0b_ref, v1w_ref, v1b_ref, o_ref):
    hdd = jnp.maximum(
        lax.dot_general(x_ref[...], v0w_ref[...], (((1,), (1,)), ((), ())),
                        precision=_HI, preferred_element_type=jnp.float32)
        + v0b_ref[...][None, :], 0.0)
    o_ref[...] = lax.dot_general(
        hdd, v1w_ref[...], (((1,), (1,)), ((), ())),
        precision=_HI, preferred_element_type=jnp.float32) \
        + v1b_ref[...][None, :]


def _tc_readout(X, V0_w, V0_b, V1_w, V1_b):
    return pl.pallas_call(
        _ro_body,
        grid=(_N // _BN,),
        in_specs=[
            pl.BlockSpec((_BN, _NH), lambda i: (i, 0)),
            pl.BlockSpec((_NH, _NH), lambda i: (0, 0)),
            pl.BlockSpec((_NH,), lambda i: (0,)),
            pl.BlockSpec((_NH, _NH), lambda i: (0, 0)),
            pl.BlockSpec((_NH,), lambda i: (0,)),
        ],
        out_specs=pl.BlockSpec((_BN, _NH), lambda i: (i, 0)),
        out_shape=jax.ShapeDtypeStruct((_N, _NH), jnp.float32),
    )(X, V0_w, V0_b, V1_w, V1_b)


def _lg_body(he_ref, p3_ref, o_ref):
    o_ref[...] = lax.dot_general(
        he_ref[...], p3_ref[...], (((1,), (1,)), ((), ())),
        precision=_HI, preferred_element_type=jnp.float32)


def _tc_logits(He, P3):
    return pl.pallas_call(
        _lg_body,
        grid=(_E // _BE,),
        in_specs=[
            pl.BlockSpec((_BE, _NE), lambda i: (i, 0)),
            pl.BlockSpec((3, _NE), lambda i: (0, 0)),
        ],
        out_specs=pl.BlockSpec((_BE, 3), lambda i: (i, 0)),
        out_shape=jax.ShapeDtypeStruct((_E, 3), jnp.float32),
    )(He, P3)


def _ht_body(he_ref, o_ref):
    o_ref[...] = he_ref[...].T


def _tc_het(He):
    return pl.pallas_call(
        _ht_body,
        grid=(_E // _BE,),
        in_specs=[pl.BlockSpec((_BE, _NE), lambda i: (i, 0))],
        out_specs=pl.BlockSpec((_NE, _BE), lambda i: (0, i)),
        out_shape=jax.ShapeDtypeStruct((_NE, _E), jnp.float32),
    )(He)


def kernel(R, S, H, node_data, Ra_data, W, Omega, W_e, Omega_e,
           B_ne, B_en, P3, V0_w, V0_b, V1_w, V1_b):
    r2 = R.reshape(_NCHUNK, _IPC, 128)
    s2 = S.reshape(_NCHUNK, _IPC, 128)
    h2 = H.reshape(_NCHUNK, _IPC, 128)

    U = _tc_u(node_data, Omega)
    Ue = _tc_ue(Ra_data, Omega_e)

    He = jnp.zeros((_E, _NE), jnp.float32)
    Y = jnp.zeros((_NC, _N, _NHH), jnp.float32)
    Z = jnp.zeros((_N, _NE), jnp.float32)
    X = jnp.zeros((_N, _NH), jnp.float32)
    for _ in range(_ITERS):
        A = _tc_edgea(He, Ue, W_e)
        He, a128, a16 = _sc_edge(r2, s2, h2, Z, Y, A)
        X, Y, Z = _tc_update(a128.reshape(_NC, _N, _NHH),
                             a16.reshape(_NC, _N, _NE), U, W, B_ne, B_en)

    x = _tc_readout(X, V0_w, V0_b, V1_w, V1_b)
    logits = _tc_logits(He, P3)
    He_T = _tc_het(He)
    return (x, He_T, logits)
